# Initial kernel scaffold; baseline (speedup 1.0000x reference)
import jax
import jax.numpy as jnp
from jax.experimental import pallas as pl


def kernel(x, Wg, bg, W1, b1, W2, b2):
    raise NotImplementedError("write your pallas kernel here")


# speedup vs baseline: 1.5768x; 1.5768x over previous
import functools

import jax
import jax.numpy as jnp
from jax import lax
from jax.experimental import pallas as pl
from jax.experimental.pallas import tpu as pltpu
from jax.experimental.pallas import tpu_sc as plsc

BT = 128


def _routing_body(T, E, NBLK, x_ref, wg_ref, bg_ref,
                  w0_ref, w1_ref, d0_ref, d1_ref, blk_ref):
    g = jnp.dot(x_ref[...], wg_ref[...], preferred_element_type=jnp.float32)
    g = g + bg_ref[...]
    ie = lax.broadcasted_iota(jnp.int32, (T, E), 1)
    v1 = jnp.max(g, axis=1, keepdims=True)
    e0 = jnp.min(jnp.where(g == v1, ie, E), axis=1, keepdims=True)
    g2 = jnp.where(ie == e0, -jnp.inf, g)
    v2 = jnp.max(g2, axis=1, keepdims=True)
    e1 = jnp.min(jnp.where(g2 == v2, ie, E), axis=1, keepdims=True)
    ex = jnp.exp(v2 - v1)
    denom = 1.0 + ex
    w0_ref[...] = 1.0 / denom
    w1_ref[...] = ex / denom

    h0 = (ie == e0).astype(jnp.float32)
    h1 = (ie == e1).astype(jnp.float32)
    h = h0 + h1
    lt = (lax.broadcasted_iota(jnp.int32, (T, T), 0)
          >= lax.broadcasted_iota(jnp.int32, (T, T), 1)).astype(jnp.float32)
    sinc = jnp.dot(lt, h, preferred_element_type=jnp.float32)
    sexc = sinc - h
    counts = jnp.sum(h, axis=0, keepdims=True)
    cnt_i = counts.astype(jnp.int32)
    pad_i = ((cnt_i + (BT - 1)) >> 7) << 7
    pad = pad_i.astype(jnp.float32)
    strict = (lax.broadcasted_iota(jnp.int32, (E, E), 0)
              < lax.broadcasted_iota(jnp.int32, (E, E), 1)).astype(jnp.float32)
    off = jnp.dot(pad, strict, preferred_element_type=jnp.float32)
    rank0 = jnp.sum(h0 * sexc, axis=1, keepdims=True)
    rank1 = jnp.sum(h1 * sexc, axis=1, keepdims=True)
    base0 = jnp.sum(h0 * off, axis=1, keepdims=True)
    base1 = jnp.sum(h1 * off, axis=1, keepdims=True)
    d0_ref[...] = (base0 + rank0).astype(jnp.int32)
    d1_ref[...] = (base1 + rank1).astype(jnp.int32)

    total = jnp.sum(pad_i)
    nvb = jnp.maximum(total >> 7, 1)
    gi = lax.broadcasted_iota(jnp.int32, (NBLK, E), 0)
    gic = jnp.minimum(gi, nvb - 1)
    offb = jnp.broadcast_to(off, (NBLK, E))
    cmp = (offb <= (gic << 7).astype(jnp.float32)).astype(jnp.float32)
    blk_ref[...] = (jnp.sum(cmp, axis=1, keepdims=True) - 1.0).astype(jnp.int32)


def _routing(x, Wg, bg, NBLK):
    T, D = x.shape
    E = Wg.shape[1]
    out_shapes = (
        jax.ShapeDtypeStruct((T, 1), jnp.float32),
        jax.ShapeDtypeStruct((T, 1), jnp.float32),
        jax.ShapeDtypeStruct((T, 1), jnp.int32),
        jax.ShapeDtypeStruct((T, 1), jnp.int32),
        jax.ShapeDtypeStruct((NBLK, 1), jnp.int32),
    )
    return pl.pallas_call(
        functools.partial(_routing_body, T, E, NBLK),
        out_shape=out_shapes,
    )(x, Wg, bg.reshape(1, E))


def _dispatch(x, d0, d1, w0, w1, P):
    T, D = x.shape
    info = plsc.get_sparse_core_info()
    NC, NS, L = info.num_cores, info.num_subcores, info.num_lanes
    NW = NC * NS
    CT = T // NW
    G = 16
    NG = CT // G
    mesh = plsc.VectorSubcoreMesh(core_axis_name="c", subcore_axis_name="s")

    @functools.partial(
        pl.kernel, mesh=mesh,
        out_type=(jax.ShapeDtypeStruct((P, D), jnp.float32),
                  jax.ShapeDtypeStruct((P, 128), jnp.float32)),
        scratch_types=[
            pltpu.VMEM((NG, G), jnp.int32),
            pltpu.VMEM((NG, G), jnp.int32),
            pltpu.VMEM((NG, G), jnp.float32),
            pltpu.VMEM((NG, G), jnp.float32),
            pltpu.VMEM((G, D), jnp.float32),
            pltpu.VMEM((G, 128), jnp.float32),
            pltpu.VMEM((G, 128), jnp.float32),
            pltpu.SemaphoreType.DMA,
        ],
    )
    def disp(x_hbm, d0_hbm, d1_hbm, w0_hbm, w1_hbm, xs_hbm, ws_hbm,
             i0_v, i1_v, w0_v, w1_v, rows_v, wr0_v, wr1_v, sem):
        wid = lax.axis_index("s") * NC + lax.axis_index("c")
        base = wid * CT
        for g in range(NG):
            pltpu.sync_copy(d0_hbm.at[pl.ds(base + g * G, G)], i0_v.at[g])
            pltpu.sync_copy(d1_hbm.at[pl.ds(base + g * G, G)], i1_v.at[g])
            pltpu.sync_copy(w0_hbm.at[pl.ds(base + g * G, G)], w0_v.at[g])
            pltpu.sync_copy(w1_hbm.at[pl.ds(base + g * G, G)], w1_v.at[g])
        for g in range(NG):
            pltpu.sync_copy(x_hbm.at[pl.ds(base + g * G, G)], rows_v)
            wv0 = w0_v[g, :]
            wv1 = w1_v[g, :]
            for j in range(G):
                wr0_v[j, pl.ds(0, L)] = jnp.broadcast_to(wv0[j], (L,))
                wr1_v[j, pl.ds(0, L)] = jnp.broadcast_to(wv1[j], (L,))
            pltpu.async_copy(rows_v, xs_hbm.at[i0_v.at[g]], sem).wait()
            pltpu.async_copy(rows_v, xs_hbm.at[i1_v.at[g]], sem).wait()
            pltpu.async_copy(wr0_v, ws_hbm.at[i0_v.at[g]], sem).wait()
            pltpu.async_copy(wr1_v, ws_hbm.at[i1_v.at[g]], sem).wait()

    return disp(x, d0, d1, w0, w1)


def _ffn_body(s_ref, x_ref, w1_ref, b1_ref, w2_ref, b2_ref, ws_ref, y_ref):
    h = jnp.dot(x_ref[...], w1_ref[0], preferred_element_type=jnp.float32)
    h = jnp.maximum(h + b1_ref[0], 0.0)
    y = jnp.dot(h, w2_ref[0], preferred_element_type=jnp.float32)
    y = y + b2_ref[0]
    y_ref[...] = y * ws_ref[:, 0:1]


def _ffn(xs, ws, W1, b1, W2, b2, blk_exp):
    P, D = xs.shape
    E, _, FF = W1.shape
    NBLK = P // BT
    grid_spec = pltpu.PrefetchScalarGridSpec(
        num_scalar_prefetch=1,
        grid=(NBLK,),
        in_specs=[
            pl.BlockSpec((BT, D), lambda g, s: (g, 0)),
            pl.BlockSpec((1, D, FF), lambda g, s: (s[g], 0, 0)),
            pl.BlockSpec((1, 1, FF), lambda g, s: (s[g], 0, 0)),
            pl.BlockSpec((1, FF, D), lambda g, s: (s[g], 0, 0)),
            pl.BlockSpec((1, 1, D), lambda g, s: (s[g], 0, 0)),
            pl.BlockSpec((BT, 128), lambda g, s: (g, 0)),
        ],
        out_specs=pl.BlockSpec((BT, D), lambda g, s: (g, 0)),
    )
    return pl.pallas_call(
        _ffn_body,
        grid_spec=grid_spec,
        out_shape=jax.ShapeDtypeStruct((P, D), jnp.float32),
    )(blk_exp, xs, W1, b1.reshape(E, 1, FF), W2, b2.reshape(E, 1, D), ws)


def _combine(ys, d0, d1, T):
    P, D = ys.shape
    info = plsc.get_sparse_core_info()
    NC, NS, L = info.num_cores, info.num_subcores, info.num_lanes
    NW = NC * NS
    CT = T // NW
    G = 16
    NG = CT // G
    mesh = plsc.VectorSubcoreMesh(core_axis_name="c", subcore_axis_name="s")

    @functools.partial(
        pl.kernel, mesh=mesh,
        out_type=jax.ShapeDtypeStruct((T, D), jnp.float32),
        scratch_types=[
            pltpu.VMEM((NG, G), jnp.int32),
            pltpu.VMEM((NG, G), jnp.int32),
            pltpu.VMEM((G, D), jnp.float32),
            pltpu.VMEM((G, D), jnp.float32),
            pltpu.SemaphoreType.DMA,
        ],
    )
    def comb(ys_hbm, d0_hbm, d1_hbm, out_hbm, i0_v, i1_v, r0_v, r1_v, sem):
        wid = lax.axis_index("s") * NC + lax.axis_index("c")
        base = wid * CT
        for g in range(NG):
            pltpu.sync_copy(d0_hbm.at[pl.ds(base + g * G, G)], i0_v.at[g])
            pltpu.sync_copy(d1_hbm.at[pl.ds(base + g * G, G)], i1_v.at[g])
        for g in range(NG):
            pltpu.async_copy(ys_hbm.at[i0_v.at[g]], r0_v, sem).wait()
            pltpu.async_copy(ys_hbm.at[i1_v.at[g]], r1_v, sem).wait()
            for j in range(G):
                def body(c, _, j=j):
                    sl = pl.ds(c * L, L)
                    r0_v[j, sl] = r0_v[j, sl] + r1_v[j, sl]
                    return 0
                lax.fori_loop(0, D // L, body, 0)
            pltpu.sync_copy(r0_v, out_hbm.at[pl.ds(base + g * G, G)])

    return comb(ys, d0, d1)


def kernel(x, Wg, bg, W1, b1, W2, b2):
    T, D = x.shape
    E = Wg.shape[1]
    A = 2 * T
    P = A + E * BT
    NBLK = P // BT

    w0, w1, d0, d1, blk_exp = _routing(x, Wg, bg, NBLK)
    d0 = d0.reshape(T)
    d1 = d1.reshape(T)
    xs, ws = _dispatch(x, d0, d1, w0.reshape(T), w1.reshape(T), P)
    ys = _ffn(xs, ws, W1, b1, W2, b2, blk_exp.reshape(NBLK))
    return _combine(ys, d0, d1, T)
